# baseline (speedup 1.0000x reference)
import functools

import jax
import jax.numpy as jnp
from jax import lax
from jax.experimental import pallas as pl
from jax.experimental.pallas import tpu as pltpu
from jax.experimental.pallas import tpu_sc as plsc

N = 10000
NE = 320000
D_IN = 128
D_HID = 128
D_OUT = 40

NW = 32
EPT = NE // NW
K = 80
NCH = EPT // K

RPW = 640
TAIL = N - 15 * RPW

_mesh = plsc.VectorSubcoreMesh(core_axis_name="c", subcore_axis_name="s")


def _zero_acc(zeros_hbm, acc, s):
    @pl.when(s < 15)
    def _():
        pltpu.sync_copy(zeros_hbm, acc.at[pl.ds(s * RPW, RPW)])

    @pl.when(s == 15)
    def _():
        pltpu.sync_copy(zeros_hbm.at[pl.ds(0, TAIL)], acc.at[pl.ds(15 * RPW, TAIL)])


def _write_out(acc, out_hbm, c, s):
    @pl.when(s < 15)
    def _():
        pltpu.sync_copy(acc.at[pl.ds(s * RPW, RPW)], out_hbm.at[c, pl.ds(s * RPW, RPW)])

    @pl.when(s == 15)
    def _():
        pltpu.sync_copy(acc.at[pl.ds(15 * RPW, TAIL)], out_hbm.at[c, pl.ds(15 * RPW, TAIL)])


DW = 16


def _make_deg_kernel():
    @functools.partial(
        pl.kernel,
        out_type=jax.ShapeDtypeStruct((2, N, DW), jnp.float32),
        mesh=_mesh,
        scratch_types=[
            pltpu.VMEM((4, K), jnp.int32),
            pltpu.VMEM((K, DW), jnp.float32),
            pltpu.VMEM_SHARED((N, DW), jnp.float32),
            pltpu.SemaphoreType.DMA,
            pltpu.SemaphoreType.DMA,
        ],
        compiler_params=pltpu.CompilerParams(use_tc_tiling_on_sc=False),
    )
    def deg_kernel(dsts_hbm, ones_hbm, zeros_hbm, out_hbm,
                   dst_v, ones_v, acc, isem, ssem):
        c = lax.axis_index("c")
        s = lax.axis_index("s")
        w = c * 16 + s
        _zero_acc(zeros_hbm, acc, s)
        pltpu.sync_copy(ones_hbm, ones_v)

        def issue_scatter(p4):
            pltpu.async_copy(ones_v, acc.at[dst_v.at[p4]], ssem, add=True)

        def wait_scatter(p4):
            pltpu.make_async_copy(ones_v, acc.at[dst_v.at[p4]], ssem).wait()

        def prefetch_idx(j, d4):
            pltpu.async_copy(dsts_hbm.at[w, j], dst_v.at[d4], isem)

        def wait_idx(d4):
            pltpu.make_async_copy(dsts_hbm.at[w, 0], dst_v.at[d4], isem).wait()

        pltpu.sync_copy(dsts_hbm.at[w, 0], dst_v.at[0])
        plsc.subcore_barrier()
        issue_scatter(0)
        prefetch_idx(1, 1)
        prefetch_idx(2, 2)
        wait_idx(1)
        issue_scatter(1)
        prefetch_idx(3, 3)

        def body(j, carry):
            p4 = lax.rem(j, 4)
            pf4 = lax.rem(j + 2, 4)
            w4 = lax.rem(j + 2, 4)
            wait_scatter(w4)
            wait_idx(p4)
            issue_scatter(p4)
            prefetch_idx(j + 2, pf4)
            return carry

        lax.fori_loop(2, NCH - 2, body, 0)

        jA = NCH - 2
        jB = NCH - 1
        wait_scatter((jA - 2) % 4)
        wait_idx(jA % 4)
        issue_scatter(jA % 4)
        wait_scatter((jB - 2) % 4)
        wait_idx(jB % 4)
        issue_scatter(jB % 4)
        wait_scatter(jA % 4)
        wait_scatter(jB % 4)

        plsc.subcore_barrier()
        _write_out(acc, out_hbm, c, s)

    return deg_kernel


def _make_agg_kernel(D, WAYS):
    SRING = 2 * WAYS
    DRING = 4 * WAYS
    @functools.partial(
        pl.kernel,
        out_type=jax.ShapeDtypeStruct((2, N, D), jnp.float32),
        mesh=_mesh,
        scratch_types=[
            pltpu.VMEM((SRING, K), jnp.int32),
            pltpu.VMEM((DRING, K), jnp.int32),
            pltpu.VMEM((SRING, K, D), jnp.float32),
            pltpu.VMEM_SHARED((N, D), jnp.float32),
        ] + [pltpu.SemaphoreType.DMA] * WAYS
          + [pltpu.SemaphoreType.DMA]
          + [pltpu.SemaphoreType.DMA] * WAYS,
        compiler_params=pltpu.CompilerParams(use_tc_tiling_on_sc=False),
    )
    def agg_kernel(hs_hbm, srcs_hbm, dsts_hbm, zeros_hbm, out_hbm,
                   src_v, dst_v, stage, acc, *sems):
        gs = sems[:WAYS]
        isem = sems[WAYS]
        ss = sems[WAYS + 1:]
        c = lax.axis_index("c")
        s = lax.axis_index("s")
        w = c * 16 + s
        _zero_acc(zeros_hbm, acc, s)

        def gather(cs, sem):
            pltpu.async_copy(hs_hbm.at[src_v.at[cs]], stage.at[cs], sem)

        def gwait(cs, sem):
            pltpu.make_async_copy(hs_hbm.at[src_v.at[cs]], stage.at[cs], sem).wait()

        def scat(cs, cd, sem):
            pltpu.async_copy(stage.at[cs], acc.at[dst_v.at[cd]], sem, add=True)

        def swait(cs, cd, sem):
            pltpu.make_async_copy(stage.at[cs], acc.at[dst_v.at[cd]], sem).wait()

        def pref(j, cs, cd):
            pltpu.async_copy(srcs_hbm.at[w, j], src_v.at[cs], isem)
            pltpu.async_copy(dsts_hbm.at[w, j], dst_v.at[cd], isem)

        def pwait(cs, cd):
            pltpu.make_async_copy(srcs_hbm.at[w, 0], src_v.at[cs], isem).wait()
            pltpu.make_async_copy(dsts_hbm.at[w, 0], dst_v.at[cd], isem).wait()

        def step(j, u, do_swait, do_gather, do_pref):
            if isinstance(j, int):
                rs, rd = j % SRING, j % DRING
                gs_, gd_ = (j + WAYS) % SRING, (j + WAYS) % DRING
                ws_, wd_ = (j - WAYS) % SRING, (j - WAYS) % DRING
                ps_, pd_ = (j + WAYS + 1) % SRING, (j + WAYS + 1) % DRING
            else:
                rs, rd = lax.rem(j, SRING), lax.rem(j, DRING)
                gs_, gd_ = lax.rem(j + WAYS, SRING), lax.rem(j + WAYS, DRING)
                ws_, wd_ = lax.rem(j + WAYS, SRING), lax.rem(j + 3 * WAYS, DRING)
                ps_, pd_ = lax.rem(j + WAYS + 1, SRING), lax.rem(j + WAYS + 1, DRING)
            gwait(rs, gs[u])
            if do_swait:
                swait(ws_, wd_, ss[u])
            scat(rs, rd, ss[u])
            if do_gather:
                pwait(gs_, gd_)
                gather(gs_, gs[u])
            if do_pref:
                pref(j + WAYS + 1, ps_, pd_)

        pltpu.sync_copy(srcs_hbm.at[w, 0], src_v.at[0])
        pltpu.sync_copy(dsts_hbm.at[w, 0], dst_v.at[0])
        plsc.subcore_barrier()
        gather(0, gs[0])
        pref(1, 1, 1)
        for jj in range(1, WAYS):
            pwait(jj, jj)
            gather(jj, gs[jj])
            pref(jj + 1, jj + 1, jj + 1)

        for jj in range(WAYS):
            step(jj, jj, False, True, True)

        LOOP_END = WAYS + ((NCH - 2 * WAYS - 1) // WAYS) * WAYS

        def body(t, carry):
            j = WAYS + t * WAYS
            for u in range(WAYS):
                step(j + u, u, True, True, True)
            return carry

        lax.fori_loop(0, (LOOP_END - WAYS) // WAYS, body, 0)

        for j in range(LOOP_END, NCH):
            u = j % WAYS
            step(j, u, True, j + WAYS < NCH, j + WAYS + 1 < NCH)

        for j in range(NCH - WAYS, NCH):
            swait(j % SRING, j % DRING, ss[j % WAYS])

        plsc.subcore_barrier()
        _write_out(acc, out_hbm, c, s)

    return agg_kernel


_deg_kernel = _make_deg_kernel()
_agg128 = _make_agg_kernel(D_HID, 2)
_agg40 = _make_agg_kernel(D_OUT, 8)

_TCB = 1000


def _tc1_body(deg_ref, x_ref, w_ref, hs_ref, dinv_ref):
    deg = (jnp.sum(deg_ref[...], axis=(0, 2)) * (1.0 / DW))[:, None] + 1.0
    dinv = lax.rsqrt(deg)
    h = jnp.dot(x_ref[...], w_ref[...], preferred_element_type=jnp.float32)
    hs_ref[...] = h * dinv
    dinv_ref[...] = dinv


def _tc1(degp, x, W1):
    grid = (N // _TCB,)
    return pl.pallas_call(
        _tc1_body,
        grid=grid,
        in_specs=[
            pl.BlockSpec((2, _TCB, DW), lambda i: (0, i, 0)),
            pl.BlockSpec((_TCB, D_IN), lambda i: (i, 0)),
            pl.BlockSpec((D_IN, D_HID), lambda i: (0, 0)),
        ],
        out_specs=[
            pl.BlockSpec((_TCB, D_HID), lambda i: (i, 0)),
            pl.BlockSpec((_TCB, 1), lambda i: (i, 0)),
        ],
        out_shape=[
            jax.ShapeDtypeStruct((N, D_HID), jnp.float32),
            jax.ShapeDtypeStruct((N, 1), jnp.float32),
        ],
    )(degp, x, W1)


def _tc2_body(agg_ref, hs1_ref, dinv_ref, b1_ref, w2_ref, hs2_ref):
    dinv = dinv_ref[...]
    o = (agg_ref[0] + agg_ref[1] + hs1_ref[...]) * dinv + b1_ref[...]
    o = jnp.maximum(o, 0.0)
    h2 = jnp.dot(o, w2_ref[...], preferred_element_type=jnp.float32)
    hs2_ref[...] = h2 * dinv


def _tc2(agg, hs1, dinv, b1, W2):
    grid = (N // _TCB,)
    return pl.pallas_call(
        _tc2_body,
        grid=grid,
        in_specs=[
            pl.BlockSpec((2, _TCB, D_HID), lambda i: (0, i, 0)),
            pl.BlockSpec((_TCB, D_HID), lambda i: (i, 0)),
            pl.BlockSpec((_TCB, 1), lambda i: (i, 0)),
            pl.BlockSpec((1, D_HID), lambda i: (0, 0)),
            pl.BlockSpec((D_HID, D_OUT), lambda i: (0, 0)),
        ],
        out_specs=pl.BlockSpec((_TCB, D_OUT), lambda i: (i, 0)),
        out_shape=jax.ShapeDtypeStruct((N, D_OUT), jnp.float32),
    )(agg, hs1, dinv, b1, W2)


def _tc3_body(agg_ref, hs2_ref, dinv_ref, b2_ref, out_ref):
    z = (agg_ref[0] + agg_ref[1] + hs2_ref[...]) * dinv_ref[...] + b2_ref[...]
    m = jnp.max(z, axis=1, keepdims=True)
    e = jnp.exp(z - m)
    lse = jnp.log(jnp.sum(e, axis=1, keepdims=True)) + m
    out_ref[...] = z - lse


def _tc3(agg, hs2, dinv, b2):
    grid = (N // _TCB,)
    return pl.pallas_call(
        _tc3_body,
        grid=grid,
        in_specs=[
            pl.BlockSpec((2, _TCB, D_OUT), lambda i: (0, i, 0)),
            pl.BlockSpec((_TCB, D_OUT), lambda i: (i, 0)),
            pl.BlockSpec((_TCB, 1), lambda i: (i, 0)),
            pl.BlockSpec((1, D_OUT), lambda i: (0, 0)),
        ],
        out_specs=pl.BlockSpec((_TCB, D_OUT), lambda i: (i, 0)),
        out_shape=jax.ShapeDtypeStruct((N, D_OUT), jnp.float32),
    )(agg, hs2, dinv, b2)


def kernel(x, edge_index, W1, b1, W2, b2):
    e = edge_index.astype(jnp.int32)
    src_r = e[0].reshape(NW, NCH, K)
    dst_r = e[1].reshape(NW, NCH, K)

    ones_kd = jnp.ones((K, DW), jnp.float32)
    zeros_d = jnp.zeros((RPW, DW), jnp.float32)
    zeros_h = jnp.zeros((RPW, D_HID), jnp.float32)
    zeros_o = jnp.zeros((RPW, D_OUT), jnp.float32)

    degp = _deg_kernel(dst_r, ones_kd, zeros_d)
    hs1, dinv = _tc1(degp, x, W1)
    agg1 = _agg128(hs1, src_r, dst_r, zeros_h)
    hs2 = _tc2(agg1, hs1, dinv, b1.reshape(1, D_HID), W2)
    agg2 = _agg40(hs2, src_r, dst_r, zeros_o)
    return _tc3(agg2, hs2, dinv, b2.reshape(1, D_OUT))
